# baseline (speedup 1.0000x reference)
import functools

import jax
import jax.numpy as jnp
from jax import lax
from jax.experimental import pallas as pl
from jax.experimental.pallas import tpu as pltpu
from jax.experimental.pallas import tpu_sc as plsc

NTIME, NBATCH, K = 100, 1024, 128
NT_PAD = 128
NC, NS, L = 2, 16, 16
NW = NC * NS
BCOLS = NBATCH // NW
NG = BCOLS // L
RS = 4 * BCOLS


def _interp_body(times_hbm, values_hbm, t_hbm, out_hbm,
                 times_s, values_s, tab, t_v, out_v, sem1, sem2, sem3):
    wid = lax.axis_index("s") * NC + lax.axis_index("c")
    b0 = pl.multiple_of(wid * BCOLS, BCOLS)

    c1 = pltpu.async_copy(times_hbm.at[:, pl.ds(b0, BCOLS)], times_s, sem1)
    c2 = pltpu.async_copy(values_hbm.at[:, pl.ds(b0, BCOLS)], values_s, sem1)
    c3 = pltpu.async_copy(t_hbm.at[:, pl.ds(b0, BCOLS)], t_v, sem2)
    c1.wait()
    c2.wait()

    lane = lax.iota(jnp.int32, L)
    inf16 = jnp.full((L,), jnp.inf, jnp.float32)

    @plsc.parallel_loop(0, NTIME - 1, unroll=4)
    def _tables(i):
        for g in range(NG):
            sl_ = pl.ds(g * L, L)
            ti = times_s[i, sl_]
            ti1 = times_s[i + 1, sl_]
            vi = values_s[i, sl_]
            vi1 = values_s[i + 1, sl_]
            slp = (vi1 - vi) / (ti1 - ti)
            base = i * RS + g * L
            tab[pl.ds(base, L)] = ti
            tab[pl.ds(base + BCOLS, L)] = vi - slp * ti
            tab[pl.ds(base + 2 * BCOLS, L)] = slp

    for g in range(NG):
        sl_ = pl.ds(g * L, L)
        t98 = times_s[NTIME - 2, sl_]
        t99 = times_s[NTIME - 1, sl_]
        v98 = values_s[NTIME - 2, sl_]
        v99 = values_s[NTIME - 1, sl_]
        slp = (v99 - v98) / (t99 - t98)
        base = (NTIME - 1) * RS + g * L
        tab[pl.ds(base, L)] = t99
        tab[pl.ds(base + BCOLS, L)] = v99 - slp * t99
    for r in range(NTIME, NT_PAD):
        for g in range(NG):
            tab[pl.ds(r * RS + g * L, L)] = inf16

    c3.wait()

    def _one_row(k):
        for g in range(NG):
            colg = lane + (g * L)
            tq = t_v[k, pl.ds(g * L, L)]

            tv = plsc.load_gather(tab, [colg + 63 * RS])
            pos = jnp.where(tv <= tq, colg + 63 * RS, colg - RS)
            for s in (32, 16, 8, 4, 2, 1):
                cand = pos + (s * RS)
                tv = plsc.load_gather(tab, [cand])
                pos = jnp.where(tv <= tq, cand, pos)

            iv = jnp.where(pos < colg, colg + (NTIME - 1) * RS, pos)
            isl = jnp.minimum(iv, colg + (NTIME - 2) * RS)
            b_at = plsc.load_gather(tab, [iv + BCOLS])
            sl_at = plsc.load_gather(tab, [isl + 2 * BCOLS])

            out_v[k, pl.ds(g * L, L)] = b_at + sl_at * tq

    @plsc.parallel_loop(0, K // 2, unroll=8)
    def _rows_lo(k):
        _one_row(k)

    o1 = pltpu.async_copy(out_v.at[pl.ds(0, K // 2)],
                          out_hbm.at[pl.ds(0, K // 2), pl.ds(b0, BCOLS)], sem3)

    @plsc.parallel_loop(K // 2, K, unroll=8)
    def _rows_hi(k):
        _one_row(k)

    o2 = pltpu.async_copy(out_v.at[pl.ds(K // 2, K // 2)],
                          out_hbm.at[pl.ds(K // 2, K // 2), pl.ds(b0, BCOLS)], sem2)
    o1.wait()
    o2.wait()


@jax.jit
def kernel(times, values, t):
    mesh = plsc.VectorSubcoreMesh(core_axis_name="c", subcore_axis_name="s")
    f = functools.partial(
        pl.kernel,
        out_type=jax.ShapeDtypeStruct((K, NBATCH), jnp.float32),
        mesh=mesh,
        compiler_params=pltpu.CompilerParams(use_tc_tiling_on_sc=False,
                                             needs_layout_passes=False,
                                             disable_bounds_checks=True,
                                             disable_semaphore_checks=True),
        scratch_types=[
            pltpu.VMEM((NTIME, BCOLS), jnp.float32),
            pltpu.VMEM((NTIME, BCOLS), jnp.float32),
            pltpu.VMEM((NT_PAD * RS,), jnp.float32),
            pltpu.VMEM((K, BCOLS), jnp.float32),
            pltpu.VMEM((K, BCOLS), jnp.float32),
            pltpu.SemaphoreType.DMA,
            pltpu.SemaphoreType.DMA,
            pltpu.SemaphoreType.DMA,
        ],
    )(_interp_body)
    return f(times, values, t)


# speedup vs baseline: 1.1858x; 1.1858x over previous
import functools

import jax
import jax.numpy as jnp
from jax import lax
from jax.experimental import pallas as pl
from jax.experimental.pallas import tpu as pltpu
from jax.experimental.pallas import tpu_sc as plsc

NTIME, NBATCH, K = 100, 1024, 128
NT_PAD = 128
NC, NS, L = 2, 16, 16
NW = NC * NS
BCOLS = NBATCH // NW
NG = BCOLS // L
RS = 4 * BCOLS


def _interp_body(times_hbm, values_hbm, t_hbm, out_hbm,
                 times_s, values_s, tab, t_v, out_v, sem1, sem2, sem3):
    wid = lax.axis_index("s") * NC + lax.axis_index("c")
    b0 = pl.multiple_of(wid * BCOLS, BCOLS)

    c1 = pltpu.async_copy(times_hbm.at[:, pl.ds(b0, BCOLS)], times_s, sem1)
    c2 = pltpu.async_copy(values_hbm.at[:, pl.ds(b0, BCOLS)], values_s, sem1)
    c3 = pltpu.async_copy(t_hbm.at[:, pl.ds(b0, BCOLS)], t_v, sem2)
    c1.wait()
    c2.wait()

    lane = lax.iota(jnp.int32, L)
    inf16 = jnp.full((L,), jnp.inf, jnp.float32)

    @plsc.parallel_loop(0, NTIME - 1, unroll=4)
    def _tables(i):
        for g in range(NG):
            sl_ = pl.ds(g * L, L)
            ti = times_s[i, sl_]
            ti1 = times_s[i + 1, sl_]
            vi = values_s[i, sl_]
            vi1 = values_s[i + 1, sl_]
            slp = (vi1 - vi) / (ti1 - ti)
            base = i * RS + g * L
            tab[pl.ds(base, L)] = ti
            tab[pl.ds(base + BCOLS, L)] = vi - slp * ti
            tab[pl.ds(base + 2 * BCOLS, L)] = slp

    for g in range(NG):
        sl_ = pl.ds(g * L, L)
        t98 = times_s[NTIME - 2, sl_]
        t99 = times_s[NTIME - 1, sl_]
        v98 = values_s[NTIME - 2, sl_]
        v99 = values_s[NTIME - 1, sl_]
        slp = (v99 - v98) / (t99 - t98)
        base = (NTIME - 1) * RS + g * L
        tab[pl.ds(base, L)] = t99
        tab[pl.ds(base + BCOLS, L)] = v99 - slp * t99
    for r in range(NTIME, NT_PAD):
        for g in range(NG):
            tab[pl.ds(r * RS + g * L, L)] = inf16

    c3.wait()

    def _one_row(k):
        for g in range(NG):
            colg = lane + (g * L)
            tq = t_v[k, pl.ds(g * L, L)]

            tv = plsc.load_gather(tab, [colg + 63 * RS])
            pos = jnp.where(tv <= tq, colg + 63 * RS, colg - RS)
            for s in (32, 16, 8, 4, 2, 1):
                cand = pos + (s * RS)
                tv = plsc.load_gather(tab, [cand])
                pos = jnp.where(tv <= tq, cand, pos)

            iv = jnp.where(pos < colg, colg + (NTIME - 1) * RS, pos)
            isl = jnp.minimum(iv, colg + (NTIME - 2) * RS)
            b_at = plsc.load_gather(tab, [iv + BCOLS])
            sl_at = plsc.load_gather(tab, [isl + 2 * BCOLS])

            out_v[k, pl.ds(g * L, L)] = b_at + sl_at * tq

    @plsc.parallel_loop(0, K // 2, unroll=4)
    def _rows_lo(k):
        _one_row(k)

    o1 = pltpu.async_copy(out_v.at[pl.ds(0, K // 2)],
                          out_hbm.at[pl.ds(0, K // 2), pl.ds(b0, BCOLS)], sem3)

    @plsc.parallel_loop(K // 2, K, unroll=4)
    def _rows_hi(k):
        _one_row(k)

    o2 = pltpu.async_copy(out_v.at[pl.ds(K // 2, K // 2)],
                          out_hbm.at[pl.ds(K // 2, K // 2), pl.ds(b0, BCOLS)], sem2)
    o1.wait()
    o2.wait()


@jax.jit
def kernel(times, values, t):
    mesh = plsc.VectorSubcoreMesh(core_axis_name="c", subcore_axis_name="s")
    f = functools.partial(
        pl.kernel,
        out_type=jax.ShapeDtypeStruct((K, NBATCH), jnp.float32),
        mesh=mesh,
        compiler_params=pltpu.CompilerParams(use_tc_tiling_on_sc=False,
                                             needs_layout_passes=False,
                                             disable_bounds_checks=True,
                                             disable_semaphore_checks=True),
        scratch_types=[
            pltpu.VMEM((NTIME, BCOLS), jnp.float32),
            pltpu.VMEM((NTIME, BCOLS), jnp.float32),
            pltpu.VMEM((NT_PAD * RS,), jnp.float32),
            pltpu.VMEM((K, BCOLS), jnp.float32),
            pltpu.VMEM((K, BCOLS), jnp.float32),
            pltpu.SemaphoreType.DMA,
            pltpu.SemaphoreType.DMA,
            pltpu.SemaphoreType.DMA,
        ],
    )(_interp_body)
    return f(times, values, t)


# speedup vs baseline: 1.2412x; 1.0467x over previous
import functools

import jax
import jax.numpy as jnp
from jax import lax
from jax.experimental import pallas as pl
from jax.experimental.pallas import tpu as pltpu
from jax.experimental.pallas import tpu_sc as plsc

NTIME, NBATCH, K = 100, 1024, 128
NT_PAD = 128
NC, NS, L = 2, 16, 16
NW = NC * NS
BCOLS = NBATCH // NW
NG = BCOLS // L
RS = 4 * BCOLS


def _interp_body(times_hbm, values_hbm, t_hbm, out_hbm,
                 times_s, values_s, tab, t_v, out_v, sem1, sem2, sem3):
    wid = lax.axis_index("s") * NC + lax.axis_index("c")
    b0 = pl.multiple_of(wid * BCOLS, BCOLS)

    c1 = pltpu.async_copy(times_hbm.at[:, pl.ds(b0, BCOLS)], times_s, sem1)
    c2 = pltpu.async_copy(values_hbm.at[:, pl.ds(b0, BCOLS)], values_s, sem1)
    c3 = pltpu.async_copy(t_hbm.at[:, pl.ds(b0, BCOLS)], t_v, sem2)
    c1.wait()
    c2.wait()

    lane = lax.iota(jnp.int32, L)
    inf16 = jnp.full((L,), jnp.inf, jnp.float32)

    @plsc.parallel_loop(0, NTIME - 1, unroll=4)
    def _tables(i):
        for g in range(NG):
            sl_ = pl.ds(g * L, L)
            ti = times_s[i, sl_]
            ti1 = times_s[i + 1, sl_]
            vi = values_s[i, sl_]
            vi1 = values_s[i + 1, sl_]
            slp = (vi1 - vi) / (ti1 - ti)
            base = i * RS + g * L
            tab[pl.ds(base, L)] = ti
            tab[pl.ds(base + BCOLS, L)] = vi - slp * ti
            tab[pl.ds(base + 2 * BCOLS, L)] = slp

    for g in range(NG):
        sl_ = pl.ds(g * L, L)
        t98 = times_s[NTIME - 2, sl_]
        t99 = times_s[NTIME - 1, sl_]
        v98 = values_s[NTIME - 2, sl_]
        v99 = values_s[NTIME - 1, sl_]
        slp = (v99 - v98) / (t99 - t98)
        base = (NTIME - 1) * RS + g * L
        tab[pl.ds(base, L)] = t99
        tab[pl.ds(base + BCOLS, L)] = v99 - slp * t99
    for r in range(NTIME, NT_PAD):
        for g in range(NG):
            tab[pl.ds(r * RS + g * L, L)] = inf16

    c3.wait()

    def _one_row(k):
        for g in range(NG):
            colg = lane + (g * L)
            tq = t_v[k, pl.ds(g * L, L)]

            tv = plsc.load_gather(tab, [colg + 63 * RS])
            pos = jnp.where(tv <= tq, colg + 63 * RS, colg - RS)
            for s in (32, 16, 8, 4, 2, 1):
                cand = pos + (s * RS)
                tv = plsc.load_gather(tab, [cand])
                pos = jnp.where(tv <= tq, cand, pos)

            iv = jnp.where(pos < colg, colg + (NTIME - 1) * RS, pos)
            isl = jnp.minimum(iv, colg + (NTIME - 2) * RS)
            b_at = plsc.load_gather(tab, [iv + BCOLS])
            sl_at = plsc.load_gather(tab, [isl + 2 * BCOLS])

            out_v[k, pl.ds(g * L, L)] = b_at + sl_at * tq

    @plsc.parallel_loop(0, K, unroll=2)
    def _rows(k):
        _one_row(k)

    pltpu.sync_copy(out_v, out_hbm.at[:, pl.ds(b0, BCOLS)])


@jax.jit
def kernel(times, values, t):
    mesh = plsc.VectorSubcoreMesh(core_axis_name="c", subcore_axis_name="s")
    f = functools.partial(
        pl.kernel,
        out_type=jax.ShapeDtypeStruct((K, NBATCH), jnp.float32),
        mesh=mesh,
        compiler_params=pltpu.CompilerParams(use_tc_tiling_on_sc=False,
                                             needs_layout_passes=False,
                                             disable_bounds_checks=True,
                                             disable_semaphore_checks=True),
        scratch_types=[
            pltpu.VMEM((NTIME, BCOLS), jnp.float32),
            pltpu.VMEM((NTIME, BCOLS), jnp.float32),
            pltpu.VMEM((NT_PAD * RS,), jnp.float32),
            pltpu.VMEM((K, BCOLS), jnp.float32),
            pltpu.VMEM((K, BCOLS), jnp.float32),
            pltpu.SemaphoreType.DMA,
            pltpu.SemaphoreType.DMA,
            pltpu.SemaphoreType.DMA,
        ],
    )(_interp_body)
    return f(times, values, t)
